# baseline (speedup 1.0000x reference)
import jax
import jax.numpy as jnp
from jax.experimental import pallas as pl
from jax.experimental.pallas import tpu as pltpu


BT = 1024
NBUF = 6


def _gate_kernel(x_hbm, wt_ref, b_ref, out_ref, buf, sems):
    nchunk = x_hbm.shape[0] // BT

    for s in range(NBUF):
        pltpu.make_async_copy(
            x_hbm.at[pl.ds(s * BT, BT), :], buf.at[s], sems.at[s]
        ).start()

    def step(c, _):
        slot = jax.lax.rem(c, NBUF)
        pltpu.make_async_copy(
            x_hbm.at[pl.ds(c * BT, BT), :], buf.at[slot], sems.at[slot]
        ).wait()
        logits = jnp.dot(buf[slot], wt_ref[...],
                         preferred_element_type=jnp.float32) + b_ref[...]
        m = jnp.max(logits, axis=-1, keepdims=True)
        e = jnp.exp(logits - m)
        out_ref[pl.ds(c * BT, BT), :] = e / jnp.sum(e, axis=-1, keepdims=True)

        nxt = c + NBUF

        @pl.when(nxt < nchunk)
        def _():
            pltpu.make_async_copy(
                x_hbm.at[pl.ds(nxt * BT, BT), :], buf.at[slot], sems.at[slot]
            ).start()

        return None

    jax.lax.fori_loop(0, nchunk, step, None)


def kernel(payload_tensor, W, b):
    B, S, D = payload_tensor.shape
    C = W.shape[0]
    T = B * S
    x2 = payload_tensor.reshape(T, D)
    wt = W.T
    b2 = b.reshape(1, C)

    out = pl.pallas_call(
        _gate_kernel,
        in_specs=[
            pl.BlockSpec(memory_space=pl.ANY),
            pl.BlockSpec(memory_space=pltpu.VMEM),
            pl.BlockSpec(memory_space=pltpu.VMEM),
        ],
        out_specs=pl.BlockSpec(memory_space=pltpu.VMEM),
        out_shape=jax.ShapeDtypeStruct((T, C), jnp.float32),
        scratch_shapes=[
            pltpu.VMEM((NBUF, BT, D), jnp.float32),
            pltpu.SemaphoreType.DMA((NBUF,)),
        ],
    )(x2, wt, b2)
    return out.reshape(B, S, C)
